# Initial kernel scaffold; baseline (speedup 1.0000x reference)
import jax
import jax.numpy as jnp
from jax.experimental import pallas as pl


def kernel(x, edge_index, edge_attr, Wl1, bl1, Wr1, br1, We1, att1, b1, Wl2, bl2, Wr2, br2, We2, att2, b2, Wo1, bo1, Wo2, bo2):
    raise NotImplementedError("write your pallas kernel here")


# speedup vs baseline: 14.0686x; 14.0686x over previous
import functools

import jax
import jax.numpy as jnp
from jax import lax
from jax.experimental import pallas as pl
from jax.experimental.pallas import tpu as pltpu
from jax.experimental.pallas import tpu_sc as plsc

N = 10000
E = 320000
D_IN = 128
D_EDGE = 16
H = 5
C = 16
HID = 80
OUT = 64
NEG = 0.2

NC = 2
NS = 16
NT = NC * NS

E2 = E + N
EPT = 10368
E2PAD = NT * EPT
BB = 64
NBATCH = EPT // BB
NGRP = BB // 16

ACC_ROWS = 10016
SLAB = ACC_ROWS // NS
WW = 2 * HID

S0_B = 80
S0_NG = S0_B // 16
S0_EPT = E // NT
S0_NB = S0_EPT // S0_B
S0_W = 32

_MESH = dict(core_axis_name="c", subcore_axis_name="s", num_cores=NC,
             num_subcores=NS)


def _iota16():
    return lax.iota(jnp.int32, 16)


def _splat16(v):
    return jnp.zeros((16,), jnp.int32) + v


def _s0_body(ea_hbm, dst_hbm, zeros_hbm, out_hbm,
             acc_sh, dst_v, ea_v, w_v, sem):
    s = lax.axis_index("s")
    cid = lax.axis_index("c")
    wid = cid * NS + s

    pltpu.sync_copy(zeros_hbm, acc_sh.at[pl.ds(s * SLAB, SLAB)])
    plsc.subcore_barrier()

    iota = _iota16()
    ones = jnp.ones((16,), jnp.float32)

    def batch(b, _):
        base = wid * S0_EPT + b * S0_B
        pltpu.sync_copy(dst_hbm.at[pl.ds(base, S0_B)], dst_v)
        pltpu.async_copy(ea_hbm.at[pl.ds(base, S0_B)], ea_v, sem).wait()

        def grp(g, _):
            rows = g * 16 + iota
            plsc.store_scatter(w_v, [rows, _splat16(D_EDGE)], ones)

            def feat(f, _):
                col = _splat16(f)
                v = plsc.load_gather(ea_v, [rows, col])
                plsc.store_scatter(w_v, [rows, col], v)
                return 0

            lax.fori_loop(0, D_EDGE, feat, 0)
            return 0

        lax.fori_loop(0, S0_NG, grp, 0)
        pltpu.sync_copy(w_v, acc_sh.at[dst_v], add=True)
        return 0

    lax.fori_loop(0, S0_NB, batch, 0)
    plsc.subcore_barrier()
    pltpu.sync_copy(acc_sh.at[pl.ds(s * SLAB, SLAB)],
                    out_hbm.at[cid, pl.ds(s * SLAB, SLAB)])


def _s0_call(edge_attr, dst, zeros0):
    return pl.kernel(
        _s0_body,
        out_type=jax.ShapeDtypeStruct((NC, ACC_ROWS, S0_W), jnp.float32),
        compiler_params=pltpu.CompilerParams(
            needs_layout_passes=False, use_tc_tiling_on_sc=False),
        mesh=plsc.VectorSubcoreMesh(**_MESH),
        scratch_types=[
            pltpu.VMEM_SHARED((ACC_ROWS, S0_W), jnp.float32),
            pltpu.VMEM((S0_B,), jnp.int32),
            pltpu.VMEM((S0_B, D_EDGE), jnp.float32),
            pltpu.VMEM((S0_B, S0_W), jnp.float32),
            pltpu.SemaphoreType.DMA,
        ],
    )(edge_attr, dst, zeros0)


def _edge_body(xl_hbm, xr_hbm, ef_hbm, src_hbm, dst_hbm, att_hbm, zeros_hbm,
               out_hbm,
               acc_sh, src_v, dst_v, xl_v, xr_v, ef_v, w_v, t_v, p_v, att_v,
               sem, sem2):
    s = lax.axis_index("s")
    cid = lax.axis_index("c")
    wid = cid * NS + s

    pltpu.sync_copy(zeros_hbm, acc_sh.at[pl.ds(s * SLAB, SLAB)])
    pltpu.sync_copy(att_hbm, att_v)
    plsc.subcore_barrier()

    iota = _iota16()

    def batch(b, _):
        base = wid * EPT + b * BB
        pltpu.sync_copy(src_hbm.at[pl.ds(base, BB)], src_v)
        pltpu.sync_copy(dst_hbm.at[pl.ds(base, BB)], dst_v)
        cp1 = pltpu.async_copy(xl_hbm.at[src_v], xl_v, sem)
        cp2 = pltpu.async_copy(xr_hbm.at[dst_v], xr_v, sem2)
        pltpu.sync_copy(ef_hbm.at[pl.ds(base, BB)], ef_v)
        cp1.wait()
        cp2.wait()

        def grp(g, _):
            rows = g * 16 + iota

            def feat(f, _):
                col = _splat16(f)
                xlv = plsc.load_gather(xl_v, [rows, col])
                xrv = plsc.load_gather(xr_v, [rows, col])
                efv = plsc.load_gather(ef_v, [rows, col])
                m = xlv + xrv + efv
                m = jnp.where(m >= 0.0, m, m * NEG)
                attv = plsc.load_gather(att_v, [col])
                t_v[pl.ds(f * 16, 16)] = m * attv
                return 0

            lax.fori_loop(0, HID, feat, 0)

            def head(h, _):
                def csum(c, acc):
                    return acc + t_v[pl.ds((h * 16 + c) * 16, 16)]

                sv = lax.fori_loop(0, C, csum, jnp.zeros((16,), jnp.float32))
                p_v[pl.ds(h * 16, 16)] = jnp.exp(sv)
                return 0

            lax.fori_loop(0, H, head, 0)

            def wfeat(f, _):
                col = _splat16(f)
                xlv = plsc.load_gather(xl_v, [rows, col])
                pv = p_v[pl.ds((f // 16) * 16, 16)]
                plsc.store_scatter(w_v, [rows, col], xlv * pv)
                plsc.store_scatter(w_v, [rows, col + HID], pv)
                return 0

            lax.fori_loop(0, HID, wfeat, 0)
            return 0

        lax.fori_loop(0, NGRP, grp, 0)
        pltpu.sync_copy(w_v, acc_sh.at[dst_v], add=True)
        return 0

    lax.fori_loop(0, NBATCH, batch, 0)
    plsc.subcore_barrier()
    pltpu.sync_copy(acc_sh.at[pl.ds(s * SLAB, SLAB)],
                    out_hbm.at[cid, pl.ds(s * SLAB, SLAB)])


def _edge_call(xlp, xrp, efc, src2, dst2, attf, zerosw):
    return pl.kernel(
        _edge_body,
        out_type=jax.ShapeDtypeStruct((NC, ACC_ROWS, WW), jnp.float32),
        compiler_params=pltpu.CompilerParams(
            needs_layout_passes=False, use_tc_tiling_on_sc=False),
        mesh=plsc.VectorSubcoreMesh(**_MESH),
        scratch_types=[
            pltpu.VMEM_SHARED((ACC_ROWS, WW), jnp.float32),
            pltpu.VMEM((BB,), jnp.int32),
            pltpu.VMEM((BB,), jnp.int32),
            pltpu.VMEM((BB, HID), jnp.float32),
            pltpu.VMEM((BB, HID), jnp.float32),
            pltpu.VMEM((BB, HID), jnp.float32),
            pltpu.VMEM((BB, WW), jnp.float32),
            pltpu.VMEM((HID * 16,), jnp.float32),
            pltpu.VMEM((H * 16,), jnp.float32),
            pltpu.VMEM((HID,), jnp.float32),
            pltpu.SemaphoreType.DMA,
            pltpu.SemaphoreType.DMA,
        ],
    )(xlp, xrp, efc, src2, dst2, attf, zerosw)


def _mm2_body(x_ref, wl_ref, bl_ref, wr_ref, br_ref, xl_ref, xr_ref):
    xv = x_ref[...]
    xl_ref[...] = (jnp.dot(xv, wl_ref[...], preferred_element_type=jnp.float32)
                   + bl_ref[...])
    xr_ref[...] = (jnp.dot(xv, wr_ref[...], preferred_element_type=jnp.float32)
                   + br_ref[...])


def _prep_call(x, wl, bl, wr, br):
    n, d = x.shape
    bn = 1000
    return pl.pallas_call(
        _mm2_body,
        grid=(n // bn,),
        in_specs=[
            pl.BlockSpec((bn, d), lambda i: (i, 0)),
            pl.BlockSpec(wl.shape, lambda i: (0, 0)),
            pl.BlockSpec((1, HID), lambda i: (0, 0)),
            pl.BlockSpec(wr.shape, lambda i: (0, 0)),
            pl.BlockSpec((1, HID), lambda i: (0, 0)),
        ],
        out_specs=[pl.BlockSpec((bn, HID), lambda i: (i, 0))] * 2,
        out_shape=[jax.ShapeDtypeStruct((n, HID), jnp.float32)] * 2,
    )(x, wl, bl.reshape(1, HID), wr, br.reshape(1, HID))


def _ef_body(ea_ref, we1_ref, we2_ref, o1_ref, o2_ref):
    ea = ea_ref[...]
    o1_ref[...] = jnp.dot(ea, we1_ref[...], preferred_element_type=jnp.float32)
    o2_ref[...] = jnp.dot(ea, we2_ref[...], preferred_element_type=jnp.float32)


def _ef_call(edge_attr, we1, we2):
    bn = 1000
    return pl.pallas_call(
        _ef_body,
        grid=(E // bn,),
        in_specs=[
            pl.BlockSpec((bn, D_EDGE), lambda i: (i, 0)),
            pl.BlockSpec((D_EDGE, HID), lambda i: (0, 0)),
            pl.BlockSpec((D_EDGE, HID), lambda i: (0, 0)),
        ],
        out_specs=[pl.BlockSpec((bn, HID), lambda i: (i, 0))] * 2,
        out_shape=[jax.ShapeDtypeStruct((E, HID), jnp.float32)] * 2,
    )(edge_attr, we1, we2)


def _loopattr_body(p_ref, we1_ref, we2_ref, o1_ref, o2_ref):
    sv = p_ref[0] + p_ref[1]
    cnt = sv[:, D_EDGE:D_EDGE + 1]
    la = sv[:, :D_EDGE] / jnp.maximum(cnt, 1.0)
    o1_ref[...] = jnp.dot(la, we1_ref[...], preferred_element_type=jnp.float32)
    o2_ref[...] = jnp.dot(la, we2_ref[...], preferred_element_type=jnp.float32)


def _loopattr_call(part0, we1, we2):
    bn = 1000
    return pl.pallas_call(
        _loopattr_body,
        grid=(N // bn,),
        in_specs=[
            pl.BlockSpec((NC, bn, S0_W), lambda i: (0, i, 0)),
            pl.BlockSpec((D_EDGE, HID), lambda i: (0, 0)),
            pl.BlockSpec((D_EDGE, HID), lambda i: (0, 0)),
        ],
        out_specs=[pl.BlockSpec((bn, HID), lambda i: (i, 0))] * 2,
        out_shape=[jax.ShapeDtypeStruct((N, HID), jnp.float32)] * 2,
    )(part0, we1, we2)


def _hval(p_ref, b_ref):
    sv = p_ref[0] + p_ref[1]
    return jnp.maximum(
        sv[:, :HID] / (sv[:, HID:] + 1e-16) + b_ref[...], 0.0)


def _comb_body(p_ref, b_ref, wl_ref, bl_ref, wr_ref, br_ref, xl_ref, xr_ref):
    hv = _hval(p_ref, b_ref)
    xl_ref[...] = (jnp.dot(hv, wl_ref[...], preferred_element_type=jnp.float32)
                   + bl_ref[...])
    xr_ref[...] = (jnp.dot(hv, wr_ref[...], preferred_element_type=jnp.float32)
                   + br_ref[...])


def _comb_call(part, b, wl, bl, wr, br):
    bn = 1000
    return pl.pallas_call(
        _comb_body,
        grid=(N // bn,),
        in_specs=[
            pl.BlockSpec((NC, bn, WW), lambda i: (0, i, 0)),
            pl.BlockSpec((1, HID), lambda i: (0, 0)),
            pl.BlockSpec((HID, HID), lambda i: (0, 0)),
            pl.BlockSpec((1, HID), lambda i: (0, 0)),
            pl.BlockSpec((HID, HID), lambda i: (0, 0)),
            pl.BlockSpec((1, HID), lambda i: (0, 0)),
        ],
        out_specs=[pl.BlockSpec((bn, HID), lambda i: (i, 0))] * 2,
        out_shape=[jax.ShapeDtypeStruct((N, HID), jnp.float32)] * 2,
    )(part, b.reshape(1, HID), wl, bl.reshape(1, HID), wr, br.reshape(1, HID))


def _final_body(p_ref, b_ref, wo1_ref, bo1_ref, wo2_ref, bo2_ref, o_ref):
    hv = _hval(p_ref, b_ref)
    t = (jnp.dot(hv, wo1_ref[...], preferred_element_type=jnp.float32)
         + bo1_ref[...])
    o_ref[...] = (jnp.dot(t, wo2_ref[...], preferred_element_type=jnp.float32)
                  + bo2_ref[...])


def _final_call(part, b, wo1, bo1, wo2, bo2):
    bn = 1000
    return pl.pallas_call(
        _final_body,
        grid=(N // bn,),
        in_specs=[
            pl.BlockSpec((NC, bn, WW), lambda i: (0, i, 0)),
            pl.BlockSpec((1, HID), lambda i: (0, 0)),
            pl.BlockSpec((HID, C), lambda i: (0, 0)),
            pl.BlockSpec((1, C), lambda i: (0, 0)),
            pl.BlockSpec((C, OUT), lambda i: (0, 0)),
            pl.BlockSpec((1, OUT), lambda i: (0, 0)),
        ],
        out_specs=pl.BlockSpec((bn, OUT), lambda i: (i, 0)),
        out_shape=jax.ShapeDtypeStruct((N, OUT), jnp.float32),
    )(part, b.reshape(1, HID), wo1, bo1.reshape(1, C), wo2,
      bo2.reshape(1, OUT))


def _pad_rows(a):
    return jnp.pad(a, ((0, ACC_ROWS - a.shape[0]), (0, 0)))


def kernel(x, edge_index, edge_attr,
           Wl1, bl1, Wr1, br1, We1, att1, b1,
           Wl2, bl2, Wr2, br2, We2, att2, b2,
           Wo1, bo1, Wo2, bo2):
    src = edge_index[0]
    dst = edge_index[1]
    loop_idx = jnp.arange(N, dtype=jnp.int32)
    padi = jnp.full((E2PAD - E2,), N, jnp.int32)
    src2 = jnp.concatenate([src, loop_idx, padi])
    dst2 = jnp.concatenate([dst, loop_idx, padi])
    zeros0 = jnp.zeros((SLAB, S0_W), jnp.float32)
    zerosw = jnp.zeros((SLAB, WW), jnp.float32)
    padf = jnp.zeros((E2PAD - E2, HID), jnp.float32)

    part0 = _s0_call(edge_attr, dst, zeros0)
    efl1, efl2 = _loopattr_call(part0, We1, We2)
    ef1, ef2 = _ef_call(edge_attr, We1, We2)
    xl1, xr1 = _prep_call(x, Wl1, bl1, Wr1, br1)

    efc1 = jnp.concatenate([ef1, efl1, padf])
    part1 = _edge_call(_pad_rows(xl1), _pad_rows(xr1), efc1, src2, dst2,
                       att1.reshape(HID), zerosw)

    xl2, xr2 = _comb_call(part1, b1, Wl2, bl2, Wr2, br2)
    efc2 = jnp.concatenate([ef2, efl2, padf])
    part2 = _edge_call(_pad_rows(xl2), _pad_rows(xr2), efc2, src2, dst2,
                       att2.reshape(HID), zerosw)

    return _final_call(part2, b2, Wo1, bo1, Wo2, bo2)


# speedup vs baseline: 16.9593x; 1.2055x over previous
import functools

import jax
import jax.numpy as jnp
from jax import lax
from jax.experimental import pallas as pl
from jax.experimental.pallas import tpu as pltpu
from jax.experimental.pallas import tpu_sc as plsc

N = 10000
E = 320000
D_IN = 128
D_EDGE = 16
H = 5
C = 16
HID = 80
OUT = 64
NEG = 0.2

NC = 2
NS = 16
NT = NC * NS

E2 = E + N
EPT = 10368
E2PAD = NT * EPT
BB = 96
NBATCH = EPT // BB
NGRP = BB // 16

ACC_ROWS = 10016
SLAB = ACC_ROWS // NS
WW = 96

S0_B = 80
S0_NG = S0_B // 16
S0_EPT = E // NT
S0_NB = S0_EPT // S0_B
S0_W = 32

_MESH = dict(core_axis_name="c", subcore_axis_name="s", num_cores=NC,
             num_subcores=NS)


def _iota16():
    return lax.iota(jnp.int32, 16)


def _splat16(v):
    return jnp.zeros((16,), jnp.int32) + v


def _s0_body(ea_hbm, dst_hbm, zeros_hbm, out_hbm,
             acc_sh, dst_v, ea_v, w_v, sem):
    s = lax.axis_index("s")
    cid = lax.axis_index("c")
    wid = cid * NS + s

    pltpu.sync_copy(zeros_hbm, acc_sh.at[pl.ds(s * SLAB, SLAB)])
    plsc.subcore_barrier()

    iota = _iota16()
    ones = jnp.ones((16,), jnp.float32)

    def batch(b, _):
        base = wid * S0_EPT + b * S0_B
        pltpu.sync_copy(dst_hbm.at[pl.ds(base, S0_B)], dst_v)
        pltpu.async_copy(ea_hbm.at[pl.ds(base, S0_B)], ea_v, sem).wait()

        def grp(g, _):
            rows = g * 16 + iota
            plsc.store_scatter(w_v, [rows, _splat16(D_EDGE)], ones)

            def feat(f, _):
                col = _splat16(f)
                v = plsc.load_gather(ea_v, [rows, col])
                plsc.store_scatter(w_v, [rows, col], v)
                return 0

            lax.fori_loop(0, D_EDGE, feat, 0)
            return 0

        lax.fori_loop(0, S0_NG, grp, 0)
        pltpu.sync_copy(w_v, acc_sh.at[dst_v], add=True)
        return 0

    lax.fori_loop(0, S0_NB, batch, 0)
    plsc.subcore_barrier()
    pltpu.sync_copy(acc_sh.at[pl.ds(s * SLAB, SLAB)],
                    out_hbm.at[cid, pl.ds(s * SLAB, SLAB)])


def _s0_call(edge_attr, dst, zeros0):
    return pl.kernel(
        _s0_body,
        out_type=jax.ShapeDtypeStruct((NC, ACC_ROWS, S0_W), jnp.float32),
        compiler_params=pltpu.CompilerParams(
            needs_layout_passes=False, use_tc_tiling_on_sc=False),
        mesh=plsc.VectorSubcoreMesh(**_MESH),
        scratch_types=[
            pltpu.VMEM_SHARED((ACC_ROWS, S0_W), jnp.float32),
            pltpu.VMEM((S0_B,), jnp.int32),
            pltpu.VMEM((S0_B, D_EDGE), jnp.float32),
            pltpu.VMEM((S0_B, S0_W), jnp.float32),
            pltpu.SemaphoreType.DMA,
        ],
    )(edge_attr, dst, zeros0)


def _edge_body(xl_hbm, xr_hbm, ef_hbm, src_hbm, dst_hbm, att_hbm, zeros_hbm,
               out_hbm,
               acc_sh, src0, src1, dst0, dst1, xl0, xl1, xr0, xr1, ef0, ef1,
               w0, w1, t_v, p_v, att_v,
               sem_si, sem_di, sem_xl, sem_xr, sem_ef, sem_w):
    s = lax.axis_index("s")
    cid = lax.axis_index("c")
    wid = cid * NS + s

    pltpu.sync_copy(zeros_hbm, acc_sh.at[pl.ds(s * SLAB, SLAB)])
    pltpu.sync_copy(att_hbm, att_v)
    plsc.subcore_barrier()

    iota = _iota16()
    bufs = ((src0, dst0, xl0, xr0, ef0, w0), (src1, dst1, xl1, xr1, ef1, w1))

    def ebase(b):
        return wid * EPT + b * BB

    def issue_idx(b, buf):
        base = ebase(b)
        pltpu.async_copy(src_hbm.at[pl.ds(base, BB)], buf[0], sem_si)
        pltpu.async_copy(dst_hbm.at[pl.ds(base, BB)], buf[1], sem_di)

    def wait_idx(buf):
        pltpu.make_async_copy(src_hbm.at[pl.ds(0, BB)], buf[0], sem_si).wait()
        pltpu.make_async_copy(dst_hbm.at[pl.ds(0, BB)], buf[1], sem_di).wait()

    def issue_rows(b, buf):
        base = ebase(b)
        pltpu.async_copy(xl_hbm.at[buf[0]], buf[2], sem_xl)
        pltpu.async_copy(xr_hbm.at[buf[1]], buf[3], sem_xr)
        pltpu.async_copy(ef_hbm.at[pl.ds(base, BB)], buf[4], sem_ef)

    def wait_rows(buf):
        pltpu.make_async_copy(xl_hbm.at[buf[0]], buf[2], sem_xl).wait()
        pltpu.make_async_copy(xr_hbm.at[buf[0]], buf[3], sem_xr).wait()
        pltpu.make_async_copy(ef_hbm.at[pl.ds(0, BB)], buf[4], sem_ef).wait()

    def wait_scatter(buf):
        pltpu.make_async_copy(buf[5], acc_sh.at[buf[1]], sem_w).wait()

    def compute(buf):
        xl_b, xr_b, ef_b, w_b = buf[2], buf[3], buf[4], buf[5]

        def grp(g, _):
            rows = g * 16 + iota

            def feat(f, _):
                col = _splat16(f)
                xlv = plsc.load_gather(xl_b, [rows, col])
                xrv = plsc.load_gather(xr_b, [rows, col])
                efv = plsc.load_gather(ef_b, [rows, col])
                m = xlv + xrv + efv
                m = jnp.where(m >= 0.0, m, m * NEG)
                attv = plsc.load_gather(att_v, [col])
                t_v[pl.ds(f * 16, 16)] = m * attv
                return 0

            lax.fori_loop(0, HID, feat, 0)

            def head(h, _):
                def csum(c, acc):
                    return acc + t_v[pl.ds((h * 16 + c) * 16, 16)]

                sv = lax.fori_loop(0, C, csum, jnp.zeros((16,), jnp.float32))
                pv = jnp.exp(sv)
                p_v[pl.ds(h * 16, 16)] = pv
                plsc.store_scatter(w_b, [rows, _splat16(HID + h)], pv)
                return 0

            lax.fori_loop(0, H, head, 0)

            def wfeat(f, _):
                col = _splat16(f)
                xlv = plsc.load_gather(xl_b, [rows, col])
                pv = p_v[pl.ds((f // 16) * 16, 16)]
                plsc.store_scatter(w_b, [rows, col], xlv * pv)
                return 0

            lax.fori_loop(0, HID, wfeat, 0)
            return 0

        lax.fori_loop(0, NGRP, grp, 0)

    def do_batch(b, cur, nxt):
        wait_rows(cur)

        @pl.when(b >= 1)
        def _():
            wait_scatter(nxt)

        @pl.when(b + 1 < NBATCH)
        def _():
            issue_idx(b + 1, nxt)

        compute(cur)

        @pl.when(b + 1 < NBATCH)
        def _():
            wait_idx(nxt)
            issue_rows(b + 1, nxt)

        pltpu.async_copy(cur[5], acc_sh.at[cur[1]], sem_w, add=True)

    pltpu.sync_copy(src_hbm.at[pl.ds(ebase(0), BB)], src0)
    pltpu.sync_copy(dst_hbm.at[pl.ds(ebase(0), BB)], dst0)
    issue_rows(0, bufs[0])

    def pair(i, _):
        do_batch(2 * i, bufs[0], bufs[1])
        do_batch(2 * i + 1, bufs[1], bufs[0])
        return 0

    lax.fori_loop(0, NBATCH // 2, pair, 0)
    wait_scatter(bufs[1])

    plsc.subcore_barrier()
    pltpu.sync_copy(acc_sh.at[pl.ds(s * SLAB, SLAB)],
                    out_hbm.at[cid, pl.ds(s * SLAB, SLAB)])


def _edge_call(xlp, xrp, efc, src2, dst2, attf, zerosw):
    return pl.kernel(
        _edge_body,
        out_type=jax.ShapeDtypeStruct((NC, ACC_ROWS, WW), jnp.float32),
        compiler_params=pltpu.CompilerParams(
            needs_layout_passes=False, use_tc_tiling_on_sc=False),
        mesh=plsc.VectorSubcoreMesh(**_MESH),
        scratch_types=[
            pltpu.VMEM_SHARED((ACC_ROWS, WW), jnp.float32),
            pltpu.VMEM((BB,), jnp.int32),
            pltpu.VMEM((BB,), jnp.int32),
            pltpu.VMEM((BB,), jnp.int32),
            pltpu.VMEM((BB,), jnp.int32),
            pltpu.VMEM((BB, HID), jnp.float32),
            pltpu.VMEM((BB, HID), jnp.float32),
            pltpu.VMEM((BB, HID), jnp.float32),
            pltpu.VMEM((BB, HID), jnp.float32),
            pltpu.VMEM((BB, HID), jnp.float32),
            pltpu.VMEM((BB, HID), jnp.float32),
            pltpu.VMEM((BB, WW), jnp.float32),
            pltpu.VMEM((BB, WW), jnp.float32),
            pltpu.VMEM((HID * 16,), jnp.float32),
            pltpu.VMEM((H * 16,), jnp.float32),
            pltpu.VMEM((HID,), jnp.float32),
            pltpu.SemaphoreType.DMA,
            pltpu.SemaphoreType.DMA,
            pltpu.SemaphoreType.DMA,
            pltpu.SemaphoreType.DMA,
            pltpu.SemaphoreType.DMA,
            pltpu.SemaphoreType.DMA,
        ],
    )(xlp, xrp, efc, src2, dst2, attf, zerosw)


def _mm2_body(x_ref, wl_ref, bl_ref, wr_ref, br_ref, xl_ref, xr_ref):
    xv = x_ref[...]
    xl_ref[...] = (jnp.dot(xv, wl_ref[...], preferred_element_type=jnp.float32)
                   + bl_ref[...])
    xr_ref[...] = (jnp.dot(xv, wr_ref[...], preferred_element_type=jnp.float32)
                   + br_ref[...])


def _prep_call(x, wl, bl, wr, br):
    n, d = x.shape
    bn = 1000
    return pl.pallas_call(
        _mm2_body,
        grid=(n // bn,),
        in_specs=[
            pl.BlockSpec((bn, d), lambda i: (i, 0)),
            pl.BlockSpec(wl.shape, lambda i: (0, 0)),
            pl.BlockSpec((1, HID), lambda i: (0, 0)),
            pl.BlockSpec(wr.shape, lambda i: (0, 0)),
            pl.BlockSpec((1, HID), lambda i: (0, 0)),
        ],
        out_specs=[pl.BlockSpec((bn, HID), lambda i: (i, 0))] * 2,
        out_shape=[jax.ShapeDtypeStruct((n, HID), jnp.float32)] * 2,
    )(x, wl, bl.reshape(1, HID), wr, br.reshape(1, HID))


def _ef_body(ea_ref, we1_ref, we2_ref, o1_ref, o2_ref):
    ea = ea_ref[...]
    o1_ref[...] = jnp.dot(ea, we1_ref[...], preferred_element_type=jnp.float32)
    o2_ref[...] = jnp.dot(ea, we2_ref[...], preferred_element_type=jnp.float32)


def _ef_call(edge_attr, we1, we2):
    bn = 1000
    return pl.pallas_call(
        _ef_body,
        grid=(E // bn,),
        in_specs=[
            pl.BlockSpec((bn, D_EDGE), lambda i: (i, 0)),
            pl.BlockSpec((D_EDGE, HID), lambda i: (0, 0)),
            pl.BlockSpec((D_EDGE, HID), lambda i: (0, 0)),
        ],
        out_specs=[pl.BlockSpec((bn, HID), lambda i: (i, 0))] * 2,
        out_shape=[jax.ShapeDtypeStruct((E, HID), jnp.float32)] * 2,
    )(edge_attr, we1, we2)


def _loopattr_body(p_ref, we1_ref, we2_ref, o1_ref, o2_ref):
    sv = p_ref[0] + p_ref[1]
    cnt = sv[:, D_EDGE:D_EDGE + 1]
    la = sv[:, :D_EDGE] / jnp.maximum(cnt, 1.0)
    o1_ref[...] = jnp.dot(la, we1_ref[...], preferred_element_type=jnp.float32)
    o2_ref[...] = jnp.dot(la, we2_ref[...], preferred_element_type=jnp.float32)


def _loopattr_call(part0, we1, we2):
    bn = 1000
    return pl.pallas_call(
        _loopattr_body,
        grid=(N // bn,),
        in_specs=[
            pl.BlockSpec((NC, bn, S0_W), lambda i: (0, i, 0)),
            pl.BlockSpec((D_EDGE, HID), lambda i: (0, 0)),
            pl.BlockSpec((D_EDGE, HID), lambda i: (0, 0)),
        ],
        out_specs=[pl.BlockSpec((bn, HID), lambda i: (i, 0))] * 2,
        out_shape=[jax.ShapeDtypeStruct((N, HID), jnp.float32)] * 2,
    )(part0, we1, we2)


def _hval(p_ref, b_ref):
    sv = p_ref[0] + p_ref[1]
    den = jnp.repeat(sv[:, HID:HID + H], C, axis=1)
    return jnp.maximum(sv[:, :HID] / (den + 1e-16) + b_ref[...], 0.0)


def _comb_body(p_ref, b_ref, wl_ref, bl_ref, wr_ref, br_ref, xl_ref, xr_ref):
    hv = _hval(p_ref, b_ref)
    xl_ref[...] = (jnp.dot(hv, wl_ref[...], preferred_element_type=jnp.float32)
                   + bl_ref[...])
    xr_ref[...] = (jnp.dot(hv, wr_ref[...], preferred_element_type=jnp.float32)
                   + br_ref[...])


def _comb_call(part, b, wl, bl, wr, br):
    bn = 1000
    return pl.pallas_call(
        _comb_body,
        grid=(N // bn,),
        in_specs=[
            pl.BlockSpec((NC, bn, WW), lambda i: (0, i, 0)),
            pl.BlockSpec((1, HID), lambda i: (0, 0)),
            pl.BlockSpec((HID, HID), lambda i: (0, 0)),
            pl.BlockSpec((1, HID), lambda i: (0, 0)),
            pl.BlockSpec((HID, HID), lambda i: (0, 0)),
            pl.BlockSpec((1, HID), lambda i: (0, 0)),
        ],
        out_specs=[pl.BlockSpec((bn, HID), lambda i: (i, 0))] * 2,
        out_shape=[jax.ShapeDtypeStruct((N, HID), jnp.float32)] * 2,
    )(part, b.reshape(1, HID), wl, bl.reshape(1, HID), wr, br.reshape(1, HID))


def _final_body(p_ref, b_ref, wo1_ref, bo1_ref, wo2_ref, bo2_ref, o_ref):
    hv = _hval(p_ref, b_ref)
    t = (jnp.dot(hv, wo1_ref[...], preferred_element_type=jnp.float32)
         + bo1_ref[...])
    o_ref[...] = (jnp.dot(t, wo2_ref[...], preferred_element_type=jnp.float32)
                  + bo2_ref[...])


def _final_call(part, b, wo1, bo1, wo2, bo2):
    bn = 1000
    return pl.pallas_call(
        _final_body,
        grid=(N // bn,),
        in_specs=[
            pl.BlockSpec((NC, bn, WW), lambda i: (0, i, 0)),
            pl.BlockSpec((1, HID), lambda i: (0, 0)),
            pl.BlockSpec((HID, C), lambda i: (0, 0)),
            pl.BlockSpec((1, C), lambda i: (0, 0)),
            pl.BlockSpec((C, OUT), lambda i: (0, 0)),
            pl.BlockSpec((1, OUT), lambda i: (0, 0)),
        ],
        out_specs=pl.BlockSpec((bn, OUT), lambda i: (i, 0)),
        out_shape=jax.ShapeDtypeStruct((N, OUT), jnp.float32),
    )(part, b.reshape(1, HID), wo1, bo1.reshape(1, C), wo2,
      bo2.reshape(1, OUT))


def _pad_rows(a):
    return jnp.pad(a, ((0, ACC_ROWS - a.shape[0]), (0, 0)))


def kernel(x, edge_index, edge_attr,
           Wl1, bl1, Wr1, br1, We1, att1, b1,
           Wl2, bl2, Wr2, br2, We2, att2, b2,
           Wo1, bo1, Wo2, bo2):
    src = edge_index[0]
    dst = edge_index[1]
    loop_idx = jnp.arange(N, dtype=jnp.int32)
    padi = jnp.full((E2PAD - E2,), N, jnp.int32)
    src2 = jnp.concatenate([src, loop_idx, padi])
    dst2 = jnp.concatenate([dst, loop_idx, padi])
    zeros0 = jnp.zeros((SLAB, S0_W), jnp.float32)
    zerosw = jnp.zeros((SLAB, WW), jnp.float32)
    padf = jnp.zeros((E2PAD - E2, HID), jnp.float32)

    part0 = _s0_call(edge_attr, dst, zeros0)
    efl1, efl2 = _loopattr_call(part0, We1, We2)
    ef1, ef2 = _ef_call(edge_attr, We1, We2)
    xl1, xr1 = _prep_call(x, Wl1, bl1, Wr1, br1)

    efc1 = jnp.concatenate([ef1, efl1, padf])
    part1 = _edge_call(_pad_rows(xl1), _pad_rows(xr1), efc1, src2, dst2,
                       att1.reshape(HID), zerosw)

    xl2, xr2 = _comb_call(part1, b1, Wl2, bl2, Wr2, br2)
    efc2 = jnp.concatenate([ef2, efl2, padf])
    part2 = _edge_call(_pad_rows(xl2), _pad_rows(xr2), efc2, src2, dst2,
                       att2.reshape(HID), zerosw)

    return _final_call(part2, b2, Wo1, bo1, Wo2, bo2)


# speedup vs baseline: 17.0900x; 1.0077x over previous
import functools

import jax
import jax.numpy as jnp
from jax import lax
from jax.experimental import pallas as pl
from jax.experimental.pallas import tpu as pltpu
from jax.experimental.pallas import tpu_sc as plsc

N = 10000
E = 320000
D_IN = 128
D_EDGE = 16
H = 5
C = 16
HID = 80
OUT = 64
NEG = 0.2

NC = 2
NS = 16
NT = NC * NS

E2 = E + N
EPT = 10368
E2PAD = NT * EPT
BB = 96
NBATCH = EPT // BB
NGRP = BB // 16

ACC_ROWS = 10016
SLAB = ACC_ROWS // NS
WW = 96

S0_B = 80
S0_NG = S0_B // 16
S0_EPT = E // NT
S0_NB = S0_EPT // S0_B
S0_W = 32

_MESH = dict(core_axis_name="c", subcore_axis_name="s", num_cores=NC,
             num_subcores=NS)


def _iota16():
    return lax.iota(jnp.int32, 16)


def _splat16(v):
    return jnp.zeros((16,), jnp.int32) + v


def _s0_body(ea_hbm, dst_hbm, zeros_hbm, out_hbm,
             acc_sh, dst_v, ea_v, w_v, sem):
    s = lax.axis_index("s")
    cid = lax.axis_index("c")
    wid = cid * NS + s

    pltpu.sync_copy(zeros_hbm, acc_sh.at[pl.ds(s * SLAB, SLAB)])
    plsc.subcore_barrier()

    iota = _iota16()
    ones = jnp.ones((16,), jnp.float32)

    def batch(b, _):
        base = wid * S0_EPT + b * S0_B
        pltpu.sync_copy(dst_hbm.at[pl.ds(base, S0_B)], dst_v)
        pltpu.async_copy(ea_hbm.at[pl.ds(base, S0_B)], ea_v, sem).wait()

        def grp(g, _):
            rows = g * 16 + iota
            plsc.store_scatter(w_v, [rows, _splat16(D_EDGE)], ones)

            def feat(f, _):
                col = _splat16(f)
                v = plsc.load_gather(ea_v, [rows, col])
                plsc.store_scatter(w_v, [rows, col], v)
                return 0

            lax.fori_loop(0, D_EDGE, feat, 0)
            return 0

        lax.fori_loop(0, S0_NG, grp, 0)
        pltpu.sync_copy(w_v, acc_sh.at[dst_v], add=True)
        return 0

    lax.fori_loop(0, S0_NB, batch, 0)
    plsc.subcore_barrier()
    pltpu.sync_copy(acc_sh.at[pl.ds(s * SLAB, SLAB)],
                    out_hbm.at[cid, pl.ds(s * SLAB, SLAB)])


def _s0_call(edge_attr, dst, zeros0):
    return pl.kernel(
        _s0_body,
        out_type=jax.ShapeDtypeStruct((NC, ACC_ROWS, S0_W), jnp.float32),
        compiler_params=pltpu.CompilerParams(
            needs_layout_passes=False, use_tc_tiling_on_sc=False),
        mesh=plsc.VectorSubcoreMesh(**_MESH),
        scratch_types=[
            pltpu.VMEM_SHARED((ACC_ROWS, S0_W), jnp.float32),
            pltpu.VMEM((S0_B,), jnp.int32),
            pltpu.VMEM((S0_B, D_EDGE), jnp.float32),
            pltpu.VMEM((S0_B, S0_W), jnp.float32),
            pltpu.SemaphoreType.DMA,
        ],
    )(edge_attr, dst, zeros0)


def _edge_body(xl_hbm, xr_hbm, ef_hbm, src_hbm, dst_hbm, att_hbm, zeros_hbm,
               out_hbm,
               acc_sh, src0, src1, dst0, dst1, xl0, xl1, xr0, xr1, ef0, ef1,
               w0, w1, t_v, p_v, att_v,
               sem_si, sem_di, sem_xl, sem_xr, sem_ef, sem_w):
    s = lax.axis_index("s")
    cid = lax.axis_index("c")
    wid = cid * NS + s

    pltpu.sync_copy(zeros_hbm, acc_sh.at[pl.ds(s * SLAB, SLAB)])
    pltpu.sync_copy(att_hbm, att_v)
    plsc.subcore_barrier()

    iota = _iota16()
    bufs = ((src0, dst0, xl0, xr0, ef0, w0), (src1, dst1, xl1, xr1, ef1, w1))

    def ebase(b):
        return wid * EPT + b * BB

    def issue_idx(b, buf):
        base = ebase(b)
        pltpu.async_copy(src_hbm.at[pl.ds(base, BB)], buf[0], sem_si)
        pltpu.async_copy(dst_hbm.at[pl.ds(base, BB)], buf[1], sem_di)

    def wait_idx(buf):
        pltpu.make_async_copy(src_hbm.at[pl.ds(0, BB)], buf[0], sem_si).wait()
        pltpu.make_async_copy(dst_hbm.at[pl.ds(0, BB)], buf[1], sem_di).wait()

    def issue_rows(b, buf):
        base = ebase(b)
        pltpu.async_copy(xl_hbm.at[buf[0]], buf[2], sem_xl)
        pltpu.async_copy(xr_hbm.at[buf[1]], buf[3], sem_xr)
        pltpu.async_copy(ef_hbm.at[pl.ds(base, BB)], buf[4], sem_ef)

    def wait_rows(buf):
        pltpu.make_async_copy(xl_hbm.at[buf[0]], buf[2], sem_xl).wait()
        pltpu.make_async_copy(xr_hbm.at[buf[0]], buf[3], sem_xr).wait()
        pltpu.make_async_copy(ef_hbm.at[pl.ds(0, BB)], buf[4], sem_ef).wait()

    def wait_scatter(buf):
        pltpu.make_async_copy(buf[5], acc_sh.at[buf[1]], sem_w).wait()

    def compute(buf):
        xl_b, xr_b, ef_b, w_b = buf[2], buf[3], buf[4], buf[5]

        def grp(g, _):
            rows = g * 16 + iota

            def feat(j, _):
                for k in range(4):
                    f = j * 4 + k
                    col = _splat16(f)
                    xlv = plsc.load_gather(xl_b, [rows, col])
                    xrv = plsc.load_gather(xr_b, [rows, col])
                    efv = plsc.load_gather(ef_b, [rows, col])
                    m = xlv + xrv + efv
                    m = jnp.where(m >= 0.0, m, m * NEG)
                    attv = plsc.load_gather(att_v, [col])
                    t_v[pl.ds(f * 16, 16)] = m * attv
                return 0

            lax.fori_loop(0, HID // 4, feat, 0)

            for h in range(H):
                ts = [t_v[pl.ds((h * 16 + c) * 16, 16)] for c in range(C)]
                while len(ts) > 1:
                    ts = [a + b for a, b in zip(ts[::2], ts[1::2])]
                pv = jnp.exp(ts[0])
                p_v[pl.ds(h * 16, 16)] = pv
                plsc.store_scatter(w_b, [rows, _splat16(HID + h)], pv)

            for h in range(H):
                pv = p_v[pl.ds(h * 16, 16)]

                def wfeat(j, _):
                    for k in range(4):
                        col = _splat16(h * 16 + j * 4 + k)
                        xlv = plsc.load_gather(xl_b, [rows, col])
                        plsc.store_scatter(w_b, [rows, col], xlv * pv)
                    return 0

                lax.fori_loop(0, 4, wfeat, 0)
            return 0

        lax.fori_loop(0, NGRP, grp, 0)

    def do_batch(b, cur, nxt):
        wait_rows(cur)

        @pl.when(b >= 1)
        def _():
            wait_scatter(nxt)

        @pl.when(b + 1 < NBATCH)
        def _():
            issue_idx(b + 1, nxt)

        compute(cur)

        @pl.when(b + 1 < NBATCH)
        def _():
            wait_idx(nxt)
            issue_rows(b + 1, nxt)

        pltpu.async_copy(cur[5], acc_sh.at[cur[1]], sem_w, add=True)

    pltpu.sync_copy(src_hbm.at[pl.ds(ebase(0), BB)], src0)
    pltpu.sync_copy(dst_hbm.at[pl.ds(ebase(0), BB)], dst0)
    issue_rows(0, bufs[0])

    def pair(i, _):
        do_batch(2 * i, bufs[0], bufs[1])
        do_batch(2 * i + 1, bufs[1], bufs[0])
        return 0

    lax.fori_loop(0, NBATCH // 2, pair, 0)
    wait_scatter(bufs[1])

    plsc.subcore_barrier()
    pltpu.sync_copy(acc_sh.at[pl.ds(s * SLAB, SLAB)],
                    out_hbm.at[cid, pl.ds(s * SLAB, SLAB)])


def _edge_call(xlp, xrp, efc, src2, dst2, attf, zerosw):
    return pl.kernel(
        _edge_body,
        out_type=jax.ShapeDtypeStruct((NC, ACC_ROWS, WW), jnp.float32),
        compiler_params=pltpu.CompilerParams(
            needs_layout_passes=False, use_tc_tiling_on_sc=False),
        mesh=plsc.VectorSubcoreMesh(**_MESH),
        scratch_types=[
            pltpu.VMEM_SHARED((ACC_ROWS, WW), jnp.float32),
            pltpu.VMEM((BB,), jnp.int32),
            pltpu.VMEM((BB,), jnp.int32),
            pltpu.VMEM((BB,), jnp.int32),
            pltpu.VMEM((BB,), jnp.int32),
            pltpu.VMEM((BB, HID), jnp.float32),
            pltpu.VMEM((BB, HID), jnp.float32),
            pltpu.VMEM((BB, HID), jnp.float32),
            pltpu.VMEM((BB, HID), jnp.float32),
            pltpu.VMEM((BB, HID), jnp.float32),
            pltpu.VMEM((BB, HID), jnp.float32),
            pltpu.VMEM((BB, WW), jnp.float32),
            pltpu.VMEM((BB, WW), jnp.float32),
            pltpu.VMEM((HID * 16,), jnp.float32),
            pltpu.VMEM((H * 16,), jnp.float32),
            pltpu.VMEM((HID,), jnp.float32),
            pltpu.SemaphoreType.DMA,
            pltpu.SemaphoreType.DMA,
            pltpu.SemaphoreType.DMA,
            pltpu.SemaphoreType.DMA,
            pltpu.SemaphoreType.DMA,
            pltpu.SemaphoreType.DMA,
        ],
    )(xlp, xrp, efc, src2, dst2, attf, zerosw)


def _mm2_body(x_ref, wl_ref, bl_ref, wr_ref, br_ref, xl_ref, xr_ref):
    xv = x_ref[...]
    xl_ref[...] = (jnp.dot(xv, wl_ref[...], preferred_element_type=jnp.float32)
                   + bl_ref[...])
    xr_ref[...] = (jnp.dot(xv, wr_ref[...], preferred_element_type=jnp.float32)
                   + br_ref[...])


def _prep_call(x, wl, bl, wr, br):
    n, d = x.shape
    bn = 1000
    return pl.pallas_call(
        _mm2_body,
        grid=(n // bn,),
        in_specs=[
            pl.BlockSpec((bn, d), lambda i: (i, 0)),
            pl.BlockSpec(wl.shape, lambda i: (0, 0)),
            pl.BlockSpec((1, HID), lambda i: (0, 0)),
            pl.BlockSpec(wr.shape, lambda i: (0, 0)),
            pl.BlockSpec((1, HID), lambda i: (0, 0)),
        ],
        out_specs=[pl.BlockSpec((bn, HID), lambda i: (i, 0))] * 2,
        out_shape=[jax.ShapeDtypeStruct((n, HID), jnp.float32)] * 2,
    )(x, wl, bl.reshape(1, HID), wr, br.reshape(1, HID))


def _ef_body(ea_ref, we1_ref, we2_ref, o1_ref, o2_ref):
    ea = ea_ref[...]
    o1_ref[...] = jnp.dot(ea, we1_ref[...], preferred_element_type=jnp.float32)
    o2_ref[...] = jnp.dot(ea, we2_ref[...], preferred_element_type=jnp.float32)


def _ef_call(edge_attr, we1, we2):
    bn = 1000
    return pl.pallas_call(
        _ef_body,
        grid=(E // bn,),
        in_specs=[
            pl.BlockSpec((bn, D_EDGE), lambda i: (i, 0)),
            pl.BlockSpec((D_EDGE, HID), lambda i: (0, 0)),
            pl.BlockSpec((D_EDGE, HID), lambda i: (0, 0)),
        ],
        out_specs=[pl.BlockSpec((bn, HID), lambda i: (i, 0))] * 2,
        out_shape=[jax.ShapeDtypeStruct((E, HID), jnp.float32)] * 2,
    )(edge_attr, we1, we2)


def _loopattr_body(p_ref, we1_ref, we2_ref, o1_ref, o2_ref):
    sv = p_ref[0] + p_ref[1]
    cnt = sv[:, D_EDGE:D_EDGE + 1]
    la = sv[:, :D_EDGE] / jnp.maximum(cnt, 1.0)
    o1_ref[...] = jnp.dot(la, we1_ref[...], preferred_element_type=jnp.float32)
    o2_ref[...] = jnp.dot(la, we2_ref[...], preferred_element_type=jnp.float32)


def _loopattr_call(part0, we1, we2):
    bn = 1000
    return pl.pallas_call(
        _loopattr_body,
        grid=(N // bn,),
        in_specs=[
            pl.BlockSpec((NC, bn, S0_W), lambda i: (0, i, 0)),
            pl.BlockSpec((D_EDGE, HID), lambda i: (0, 0)),
            pl.BlockSpec((D_EDGE, HID), lambda i: (0, 0)),
        ],
        out_specs=[pl.BlockSpec((bn, HID), lambda i: (i, 0))] * 2,
        out_shape=[jax.ShapeDtypeStruct((N, HID), jnp.float32)] * 2,
    )(part0, we1, we2)


def _hval(p_ref, b_ref):
    sv = p_ref[0] + p_ref[1]
    den = jnp.repeat(sv[:, HID:HID + H], C, axis=1)
    return jnp.maximum(sv[:, :HID] / (den + 1e-16) + b_ref[...], 0.0)


def _comb_body(p_ref, b_ref, wl_ref, bl_ref, wr_ref, br_ref, xl_ref, xr_ref):
    hv = _hval(p_ref, b_ref)
    xl_ref[...] = (jnp.dot(hv, wl_ref[...], preferred_element_type=jnp.float32)
                   + bl_ref[...])
    xr_ref[...] = (jnp.dot(hv, wr_ref[...], preferred_element_type=jnp.float32)
                   + br_ref[...])


def _comb_call(part, b, wl, bl, wr, br):
    bn = 1000
    return pl.pallas_call(
        _comb_body,
        grid=(N // bn,),
        in_specs=[
            pl.BlockSpec((NC, bn, WW), lambda i: (0, i, 0)),
            pl.BlockSpec((1, HID), lambda i: (0, 0)),
            pl.BlockSpec((HID, HID), lambda i: (0, 0)),
            pl.BlockSpec((1, HID), lambda i: (0, 0)),
            pl.BlockSpec((HID, HID), lambda i: (0, 0)),
            pl.BlockSpec((1, HID), lambda i: (0, 0)),
        ],
        out_specs=[pl.BlockSpec((bn, HID), lambda i: (i, 0))] * 2,
        out_shape=[jax.ShapeDtypeStruct((N, HID), jnp.float32)] * 2,
    )(part, b.reshape(1, HID), wl, bl.reshape(1, HID), wr, br.reshape(1, HID))


def _final_body(p_ref, b_ref, wo1_ref, bo1_ref, wo2_ref, bo2_ref, o_ref):
    hv = _hval(p_ref, b_ref)
    t = (jnp.dot(hv, wo1_ref[...], preferred_element_type=jnp.float32)
         + bo1_ref[...])
    o_ref[...] = (jnp.dot(t, wo2_ref[...], preferred_element_type=jnp.float32)
                  + bo2_ref[...])


def _final_call(part, b, wo1, bo1, wo2, bo2):
    bn = 1000
    return pl.pallas_call(
        _final_body,
        grid=(N // bn,),
        in_specs=[
            pl.BlockSpec((NC, bn, WW), lambda i: (0, i, 0)),
            pl.BlockSpec((1, HID), lambda i: (0, 0)),
            pl.BlockSpec((HID, C), lambda i: (0, 0)),
            pl.BlockSpec((1, C), lambda i: (0, 0)),
            pl.BlockSpec((C, OUT), lambda i: (0, 0)),
            pl.BlockSpec((1, OUT), lambda i: (0, 0)),
        ],
        out_specs=pl.BlockSpec((bn, OUT), lambda i: (i, 0)),
        out_shape=jax.ShapeDtypeStruct((N, OUT), jnp.float32),
    )(part, b.reshape(1, HID), wo1, bo1.reshape(1, C), wo2,
      bo2.reshape(1, OUT))


def _pad_rows(a):
    return jnp.pad(a, ((0, ACC_ROWS - a.shape[0]), (0, 0)))


def kernel(x, edge_index, edge_attr,
           Wl1, bl1, Wr1, br1, We1, att1, b1,
           Wl2, bl2, Wr2, br2, We2, att2, b2,
           Wo1, bo1, Wo2, bo2):
    src = edge_index[0]
    dst = edge_index[1]
    loop_idx = jnp.arange(N, dtype=jnp.int32)
    padi = jnp.full((E2PAD - E2,), N, jnp.int32)
    src2 = jnp.concatenate([src, loop_idx, padi])
    dst2 = jnp.concatenate([dst, loop_idx, padi])
    zeros0 = jnp.zeros((SLAB, S0_W), jnp.float32)
    zerosw = jnp.zeros((SLAB, WW), jnp.float32)
    padf = jnp.zeros((E2PAD - E2, HID), jnp.float32)

    part0 = _s0_call(edge_attr, dst, zeros0)
    efl1, efl2 = _loopattr_call(part0, We1, We2)
    ef1, ef2 = _ef_call(edge_attr, We1, We2)
    xl1, xr1 = _prep_call(x, Wl1, bl1, Wr1, br1)

    efc1 = jnp.concatenate([ef1, efl1, padf])
    part1 = _edge_call(_pad_rows(xl1), _pad_rows(xr1), efc1, src2, dst2,
                       att1.reshape(HID), zerosw)

    xl2, xr2 = _comb_call(part1, b1, Wl2, bl2, Wr2, br2)
    efc2 = jnp.concatenate([ef2, efl2, padf])
    part2 = _edge_call(_pad_rows(xl2), _pad_rows(xr2), efc2, src2, dst2,
                       att2.reshape(HID), zerosw)

    return _final_call(part2, b2, Wo1, bo1, Wo2, bo2)


# speedup vs baseline: 17.7051x; 1.0360x over previous
import functools

import jax
import jax.numpy as jnp
from jax import lax
from jax.experimental import pallas as pl
from jax.experimental.pallas import tpu as pltpu
from jax.experimental.pallas import tpu_sc as plsc

N = 10000
E = 320000
D_IN = 128
D_EDGE = 16
H = 5
C = 16
HID = 80
OUT = 64
NEG = 0.2

NC = 2
NS = 16
NT = NC * NS

E2 = E + N
EPT = 10368
E2PAD = NT * EPT
BB = 96
NBATCH = EPT // BB
NGRP = BB // 16

ACC_ROWS = 10016
SLAB = ACC_ROWS // NS
WW = 96

S0_B = 80
S0_NG = S0_B // 16
S0_EPT = E // NT
S0_NB = S0_EPT // S0_B
S0_W = 32

_MESH = dict(core_axis_name="c", subcore_axis_name="s", num_cores=NC,
             num_subcores=NS)


def _iota16():
    return lax.iota(jnp.int32, 16)


def _splat16(v):
    return jnp.zeros((16,), jnp.int32) + v


def _s0_body(ea_hbm, dst_hbm, zeros_hbm, out_hbm,
             acc_sh, dst_v, ea_v, w_v, sem):
    s = lax.axis_index("s")
    cid = lax.axis_index("c")
    wid = cid * NS + s

    pltpu.sync_copy(zeros_hbm, acc_sh.at[pl.ds(s * SLAB, SLAB)])
    plsc.subcore_barrier()

    iota = _iota16()
    ones = jnp.ones((16,), jnp.float32)

    def batch(b, _):
        base = wid * S0_EPT + b * S0_B
        pltpu.sync_copy(dst_hbm.at[pl.ds(base, S0_B)], dst_v)
        pltpu.async_copy(ea_hbm.at[pl.ds(base, S0_B)], ea_v, sem).wait()

        def grp(g, _):
            rows = g * 16 + iota
            plsc.store_scatter(w_v, [rows, _splat16(D_EDGE)], ones)

            def feat(f, _):
                col = _splat16(f)
                v = plsc.load_gather(ea_v, [rows, col])
                plsc.store_scatter(w_v, [rows, col], v)
                return 0

            lax.fori_loop(0, D_EDGE, feat, 0)
            return 0

        lax.fori_loop(0, S0_NG, grp, 0)
        pltpu.sync_copy(w_v, acc_sh.at[dst_v], add=True)
        return 0

    lax.fori_loop(0, S0_NB, batch, 0)
    plsc.subcore_barrier()
    pltpu.sync_copy(acc_sh.at[pl.ds(s * SLAB, SLAB)],
                    out_hbm.at[cid, pl.ds(s * SLAB, SLAB)])


def _s0_call(edge_attr, dst, zeros0):
    return pl.kernel(
        _s0_body,
        out_type=jax.ShapeDtypeStruct((NC, ACC_ROWS, S0_W), jnp.float32),
        compiler_params=pltpu.CompilerParams(
            needs_layout_passes=False, use_tc_tiling_on_sc=False),
        mesh=plsc.VectorSubcoreMesh(**_MESH),
        scratch_types=[
            pltpu.VMEM_SHARED((ACC_ROWS, S0_W), jnp.float32),
            pltpu.VMEM((S0_B,), jnp.int32),
            pltpu.VMEM((S0_B, D_EDGE), jnp.float32),
            pltpu.VMEM((S0_B, S0_W), jnp.float32),
            pltpu.SemaphoreType.DMA,
        ],
    )(edge_attr, dst, zeros0)


def _edge_body(xl_hbm, xr_hbm, ef_hbm, src_hbm, dst_hbm, att_hbm, zeros_hbm,
               out_hbm,
               acc_sh, src0, src1, dst0, dst1, xl0, xl1, xr0, xr1, ef0, ef1,
               w0, w1, t_v, p_v, att_v,
               sem_si, sem_di, sem_xl, sem_xr, sem_ef, sem_w):
    s = lax.axis_index("s")
    cid = lax.axis_index("c")
    wid = cid * NS + s

    pltpu.sync_copy(zeros_hbm, acc_sh.at[pl.ds(s * SLAB, SLAB)])
    pltpu.sync_copy(att_hbm, att_v)
    plsc.subcore_barrier()

    iota = _iota16()
    bufs = ((src0, dst0, xl0, xr0, ef0, w0), (src1, dst1, xl1, xr1, ef1, w1))

    def ebase(b):
        return wid * EPT + b * BB

    def issue_idx(b, buf):
        base = ebase(b)
        pltpu.async_copy(src_hbm.at[pl.ds(base, BB)], buf[0], sem_si)
        pltpu.async_copy(dst_hbm.at[pl.ds(base, BB)], buf[1], sem_di)

    def wait_idx(buf):
        pltpu.make_async_copy(src_hbm.at[pl.ds(0, BB)], buf[0], sem_si).wait()
        pltpu.make_async_copy(dst_hbm.at[pl.ds(0, BB)], buf[1], sem_di).wait()

    def issue_rows(b, buf):
        base = ebase(b)
        pltpu.async_copy(xl_hbm.at[buf[0]], buf[2], sem_xl)
        pltpu.async_copy(xr_hbm.at[buf[1]], buf[3], sem_xr)
        pltpu.async_copy(ef_hbm.at[pl.ds(base, BB)], buf[4], sem_ef)

    def wait_rows(buf):
        pltpu.make_async_copy(xl_hbm.at[buf[0]], buf[2], sem_xl).wait()
        pltpu.make_async_copy(xr_hbm.at[buf[0]], buf[3], sem_xr).wait()
        pltpu.make_async_copy(ef_hbm.at[pl.ds(0, BB)], buf[4], sem_ef).wait()

    def wait_scatter(buf):
        pltpu.make_async_copy(buf[5], acc_sh.at[buf[1]], sem_w).wait()

    def compute(buf):
        xl_b, xr_b, ef_b, w_b = buf[2], buf[3], buf[4], buf[5]

        def grp(g, _):
            rows = g * 16 + iota

            def feat(j, _):
                for k in range(4):
                    f = j * 4 + k
                    col = _splat16(f)
                    xlv = plsc.load_gather(xl_b, [rows, col])
                    xrv = plsc.load_gather(xr_b, [rows, col])
                    efv = plsc.load_gather(ef_b, [rows, col])
                    m = xlv + xrv + efv
                    m = jnp.where(m >= 0.0, m, m * NEG)
                    attv = plsc.load_gather(att_v, [col])
                    t_v[pl.ds(f * 16, 16)] = m * attv
                return 0

            lax.fori_loop(0, HID // 4, feat, 0)

            for h in range(H):
                ts = [t_v[pl.ds((h * 16 + c) * 16, 16)] for c in range(C)]
                while len(ts) > 1:
                    ts = [a + b for a, b in zip(ts[::2], ts[1::2])]
                pv = jnp.exp(ts[0])
                p_v[pl.ds(h * 16, 16)] = pv
                plsc.store_scatter(w_b, [rows, _splat16(HID + h)], pv)

            for h in range(H):
                pv = p_v[pl.ds(h * 16, 16)]

                def wfeat(j, _):
                    for k in range(4):
                        col = _splat16(h * 16 + j * 4 + k)
                        xlv = plsc.load_gather(xl_b, [rows, col])
                        plsc.store_scatter(w_b, [rows, col], xlv * pv)
                    return 0

                lax.fori_loop(0, 4, wfeat, 0)
            return 0

        lax.fori_loop(0, NGRP, grp, 0)

    def do_batch(b, cur, nxt):
        wait_rows(cur)

        @pl.when(b >= 1)
        def _():
            wait_scatter(nxt)

        @pl.when(b + 1 < NBATCH)
        def _():
            issue_idx(b + 1, nxt)

        compute(cur)

        @pl.when(b + 1 < NBATCH)
        def _():
            wait_idx(nxt)
            issue_rows(b + 1, nxt)

        pltpu.async_copy(cur[5], acc_sh.at[cur[1]], sem_w, add=True)

    pltpu.sync_copy(src_hbm.at[pl.ds(ebase(0), BB)], src0)
    pltpu.sync_copy(dst_hbm.at[pl.ds(ebase(0), BB)], dst0)
    issue_rows(0, bufs[0])

    def pair(i, _):
        do_batch(2 * i, bufs[0], bufs[1])
        do_batch(2 * i + 1, bufs[1], bufs[0])
        return 0

    lax.fori_loop(0, NBATCH // 2, pair, 0)
    wait_scatter(bufs[1])

    plsc.subcore_barrier()
    pltpu.sync_copy(acc_sh.at[pl.ds(s * SLAB, SLAB)],
                    out_hbm.at[cid, pl.ds(s * SLAB, SLAB)])


def _edge_call(xlp, xrp, efc, src2, dst2, attf, zerosw):
    return pl.kernel(
        _edge_body,
        out_type=jax.ShapeDtypeStruct((NC, ACC_ROWS, WW), jnp.float32),
        compiler_params=pltpu.CompilerParams(
            needs_layout_passes=False, use_tc_tiling_on_sc=False),
        mesh=plsc.VectorSubcoreMesh(**_MESH),
        scratch_types=[
            pltpu.VMEM_SHARED((ACC_ROWS, WW), jnp.float32),
            pltpu.VMEM((BB,), jnp.int32),
            pltpu.VMEM((BB,), jnp.int32),
            pltpu.VMEM((BB,), jnp.int32),
            pltpu.VMEM((BB,), jnp.int32),
            pltpu.VMEM((BB, HID), jnp.float32),
            pltpu.VMEM((BB, HID), jnp.float32),
            pltpu.VMEM((BB, HID), jnp.float32),
            pltpu.VMEM((BB, HID), jnp.float32),
            pltpu.VMEM((BB, HID), jnp.float32),
            pltpu.VMEM((BB, HID), jnp.float32),
            pltpu.VMEM((BB, WW), jnp.float32),
            pltpu.VMEM((BB, WW), jnp.float32),
            pltpu.VMEM((HID * 16,), jnp.float32),
            pltpu.VMEM((H * 16,), jnp.float32),
            pltpu.VMEM((HID,), jnp.float32),
            pltpu.SemaphoreType.DMA,
            pltpu.SemaphoreType.DMA,
            pltpu.SemaphoreType.DMA,
            pltpu.SemaphoreType.DMA,
            pltpu.SemaphoreType.DMA,
            pltpu.SemaphoreType.DMA,
        ],
    )(xlp, xrp, efc, src2, dst2, attf, zerosw)


def _mm2_body(x_ref, wl_ref, bl_ref, wr_ref, br_ref, xl_ref, xr_ref):
    xv = x_ref[...]
    xl_ref[...] = (jnp.dot(xv, wl_ref[...], preferred_element_type=jnp.float32)
                   + bl_ref[...])
    xr_ref[...] = (jnp.dot(xv, wr_ref[...], preferred_element_type=jnp.float32)
                   + br_ref[...])


def _prep_call(x, wl, bl, wr, br):
    n, d = x.shape
    bn = 1000
    return pl.pallas_call(
        _mm2_body,
        grid=(n // bn,),
        in_specs=[
            pl.BlockSpec((bn, d), lambda i: (i, 0)),
            pl.BlockSpec(wl.shape, lambda i: (0, 0)),
            pl.BlockSpec((1, HID), lambda i: (0, 0)),
            pl.BlockSpec(wr.shape, lambda i: (0, 0)),
            pl.BlockSpec((1, HID), lambda i: (0, 0)),
        ],
        out_specs=[pl.BlockSpec((bn, HID), lambda i: (i, 0))] * 2,
        out_shape=[jax.ShapeDtypeStruct((ACC_ROWS, HID), jnp.float32)] * 2,
    )(x, wl, bl.reshape(1, HID), wr, br.reshape(1, HID))


def _ef_body(ea_ref, we1_ref, we2_ref, o1_ref, o2_ref):
    ea = ea_ref[...]
    o1_ref[...] = jnp.dot(ea, we1_ref[...], preferred_element_type=jnp.float32)
    o2_ref[...] = jnp.dot(ea, we2_ref[...], preferred_element_type=jnp.float32)


def _ef_call(edge_attr, we1, we2):
    bn = 1000
    return pl.pallas_call(
        _ef_body,
        grid=(E // bn,),
        in_specs=[
            pl.BlockSpec((bn, D_EDGE), lambda i: (i, 0)),
            pl.BlockSpec((D_EDGE, HID), lambda i: (0, 0)),
            pl.BlockSpec((D_EDGE, HID), lambda i: (0, 0)),
        ],
        out_specs=[pl.BlockSpec((bn, HID), lambda i: (i, 0))] * 2,
        out_shape=[jax.ShapeDtypeStruct((E2PAD, HID), jnp.float32)] * 2,
    )(edge_attr, we1, we2)


def _loopattr_body(p_ref, we1_ref, we2_ref, e1_ref, e2_ref, o1_ref, o2_ref):
    sv = p_ref[0] + p_ref[1]
    cnt = sv[:, D_EDGE:D_EDGE + 1]
    la = sv[:, :D_EDGE] / jnp.maximum(cnt, 1.0)
    o1_ref[...] = jnp.dot(la, we1_ref[...], preferred_element_type=jnp.float32)
    o2_ref[...] = jnp.dot(la, we2_ref[...], preferred_element_type=jnp.float32)


def _loopattr_call(part0, we1, we2, ef1, ef2):
    bn = 1000
    return pl.pallas_call(
        _loopattr_body,
        grid=(N // bn,),
        in_specs=[
            pl.BlockSpec((NC, bn, S0_W), lambda i: (0, i, 0)),
            pl.BlockSpec((D_EDGE, HID), lambda i: (0, 0)),
            pl.BlockSpec((D_EDGE, HID), lambda i: (0, 0)),
            pl.BlockSpec((bn, HID), lambda i: (E // bn + i, 0)),
            pl.BlockSpec((bn, HID), lambda i: (E // bn + i, 0)),
        ],
        out_specs=[pl.BlockSpec((bn, HID), lambda i: (E // bn + i, 0))] * 2,
        out_shape=[jax.ShapeDtypeStruct((E2PAD, HID), jnp.float32)] * 2,
        input_output_aliases={3: 0, 4: 1},
    )(part0, we1, we2, ef1, ef2)


def _hval(p_ref, b_ref):
    sv = p_ref[0] + p_ref[1]
    den = jnp.repeat(sv[:, HID:HID + H], C, axis=1)
    return jnp.maximum(sv[:, :HID] / (den + 1e-16) + b_ref[...], 0.0)


def _comb_body(p_ref, b_ref, wl_ref, bl_ref, wr_ref, br_ref, xl_ref, xr_ref):
    hv = _hval(p_ref, b_ref)
    xl_ref[...] = (jnp.dot(hv, wl_ref[...], preferred_element_type=jnp.float32)
                   + bl_ref[...])
    xr_ref[...] = (jnp.dot(hv, wr_ref[...], preferred_element_type=jnp.float32)
                   + br_ref[...])


def _comb_call(part, b, wl, bl, wr, br):
    bn = 1000
    return pl.pallas_call(
        _comb_body,
        grid=(N // bn,),
        in_specs=[
            pl.BlockSpec((NC, bn, WW), lambda i: (0, i, 0)),
            pl.BlockSpec((1, HID), lambda i: (0, 0)),
            pl.BlockSpec((HID, HID), lambda i: (0, 0)),
            pl.BlockSpec((1, HID), lambda i: (0, 0)),
            pl.BlockSpec((HID, HID), lambda i: (0, 0)),
            pl.BlockSpec((1, HID), lambda i: (0, 0)),
        ],
        out_specs=[pl.BlockSpec((bn, HID), lambda i: (i, 0))] * 2,
        out_shape=[jax.ShapeDtypeStruct((ACC_ROWS, HID), jnp.float32)] * 2,
    )(part, b.reshape(1, HID), wl, bl.reshape(1, HID), wr, br.reshape(1, HID))


def _final_body(p_ref, b_ref, wo1_ref, bo1_ref, wo2_ref, bo2_ref, o_ref):
    hv = _hval(p_ref, b_ref)
    t = (jnp.dot(hv, wo1_ref[...], preferred_element_type=jnp.float32)
         + bo1_ref[...])
    o_ref[...] = (jnp.dot(t, wo2_ref[...], preferred_element_type=jnp.float32)
                  + bo2_ref[...])


def _final_call(part, b, wo1, bo1, wo2, bo2):
    bn = 1000
    return pl.pallas_call(
        _final_body,
        grid=(N // bn,),
        in_specs=[
            pl.BlockSpec((NC, bn, WW), lambda i: (0, i, 0)),
            pl.BlockSpec((1, HID), lambda i: (0, 0)),
            pl.BlockSpec((HID, C), lambda i: (0, 0)),
            pl.BlockSpec((1, C), lambda i: (0, 0)),
            pl.BlockSpec((C, OUT), lambda i: (0, 0)),
            pl.BlockSpec((1, OUT), lambda i: (0, 0)),
        ],
        out_specs=pl.BlockSpec((bn, OUT), lambda i: (i, 0)),
        out_shape=jax.ShapeDtypeStruct((N, OUT), jnp.float32),
    )(part, b.reshape(1, HID), wo1, bo1.reshape(1, C), wo2,
      bo2.reshape(1, OUT))


def kernel(x, edge_index, edge_attr,
           Wl1, bl1, Wr1, br1, We1, att1, b1,
           Wl2, bl2, Wr2, br2, We2, att2, b2,
           Wo1, bo1, Wo2, bo2):
    src = edge_index[0]
    dst = edge_index[1]
    loop_idx = jnp.arange(N, dtype=jnp.int32)
    padi = jnp.full((E2PAD - E2,), N, jnp.int32)
    src2 = jnp.concatenate([src, loop_idx, padi])
    dst2 = jnp.concatenate([dst, loop_idx, padi])
    zeros0 = jnp.zeros((SLAB, S0_W), jnp.float32)
    zerosw = jnp.zeros((SLAB, WW), jnp.float32)

    part0 = _s0_call(edge_attr, dst, zeros0)
    ef1, ef2 = _ef_call(edge_attr, We1, We2)
    efc1, efc2 = _loopattr_call(part0, We1, We2, ef1, ef2)
    xl1, xr1 = _prep_call(x, Wl1, bl1, Wr1, br1)

    part1 = _edge_call(xl1, xr1, efc1, src2, dst2, att1.reshape(HID), zerosw)

    xl2, xr2 = _comb_call(part1, b1, Wl2, bl2, Wr2, br2)
    part2 = _edge_call(xl2, xr2, efc2, src2, dst2, att2.reshape(HID), zerosw)

    return _final_call(part2, b2, Wo1, bo1, Wo2, bo2)
